# baseline (speedup 1.0000x reference)
import functools

import jax
import jax.numpy as jnp
from jax import lax
from jax.experimental import pallas as pl
from jax.experimental.pallas import tpu as pltpu
from jax.experimental.pallas import tpu_sc as plsc

E = 16
TOP_K = 8
D = 2048
F = 1408
B = 32
NF = 2
C = F // NF
DC = D // NF


def _route_sc_kernel(rl_hbm, out_hbm, logit_v, w_v):
    wid = lax.axis_index("s") * 2 + lax.axis_index("c")
    pltpu.sync_copy(rl_hbm.at[wid], logit_v)
    lg = logit_v[...]
    ids = lax.iota(jnp.int32, 16)

    def _shuf(v, k):
        return v.at[ids ^ k].get(mode="promise_in_bounds")

    def _ared(v, op):
        for k in (1, 2, 4, 8):
            v = op(v, _shuf(v, k))
        return v

    vals = lg
    sel = ids < 0
    for _ in range(TOP_K):
        m = _ared(vals, jnp.maximum)
        first = _ared(jnp.where(vals == m, ids, E), jnp.minimum)
        hit = ids == first
        sel = sel | hit
        vals = jnp.where(hit, -jnp.inf, vals)
    ew = jnp.where(sel, jnp.exp(lg - _ared(lg, jnp.maximum)), 0.0)
    w_v[...] = ew / _ared(ew, jnp.add)
    pltpu.sync_copy(w_v, out_hbm.at[wid])


@functools.partial(jax.jit, static_argnames=())
def _route_sc(router_logits):
    mesh = plsc.VectorSubcoreMesh(core_axis_name="c", subcore_axis_name="s")
    return pl.kernel(
        _route_sc_kernel,
        mesh=mesh,
        out_type=jax.ShapeDtypeStruct((B, E), jnp.float32),
        scratch_types=[
            pltpu.VMEM((E,), jnp.float32),
            pltpu.VMEM((E,), jnp.float32),
        ],
    )(router_logits)


def _ffn_kernel(cb_ref, xt_ref, w1_ref, w1u_ref, w2_ref, out_ref,
                xt_v, combt, acct, mixa, mixb):
    e = pl.program_id(0)
    f = pl.program_id(1)

    @pl.when((e == 0) & (f == 0))
    def _init():
        xt_v[:, :] = xt_ref[:, :]
        acct[:, :] = jnp.zeros((D, B), jnp.float32)
        combt[:, :] = cb_ref[:, :].T

    @pl.when(e < E)
    def _gate_up():
        xtb = xt_v[:, :]
        gt = jax.lax.dot_general(w1_ref[0], xtb, (((1,), (0,)), ((), ())),
                                 preferred_element_type=jnp.float32)
        ut = jax.lax.dot_general(w1u_ref[0], xtb, (((1,), (0,)), ((), ())),
                                 preferred_element_type=jnp.float32)
        cw = combt[pl.ds(e, 1), :]
        mt = gt * jax.lax.logistic(gt) * ut * cw

        @pl.when(e % 2 == 0)
        def _():
            mixa[pl.ds(f * C, C), :] = mt

        @pl.when(e % 2 == 1)
        def _():
            mixb[pl.ds(f * C, C), :] = mt

    @pl.when(e > 0)
    def _down():
        @pl.when(e % 2 == 1)
        def _():
            acct[pl.ds(f * DC, DC), :] += jax.lax.dot_general(
                w2_ref[0], mixa[:, :], (((1,), (0,)), ((), ())),
                preferred_element_type=jnp.float32)

        @pl.when(e % 2 == 0)
        def _():
            acct[pl.ds(f * DC, DC), :] += jax.lax.dot_general(
                w2_ref[0], mixb[:, :], (((1,), (0,)), ((), ())),
                preferred_element_type=jnp.float32)

    @pl.when((e == E) & (f == NF - 1))
    def _fin():
        out_ref[:, :] = acct[:, :]


def kernel(x, router_logits, w1, w1_up, w2):
    if x.ndim == 2:
        x = x[:, None, :]
    curr = x[:, -1, :]
    comb = _route_sc(router_logits)
    outt = pl.pallas_call(
        _ffn_kernel,
        grid=(E + 1, NF),
        in_specs=[
            pl.BlockSpec((B, E), lambda e, f: (0, 0)),
            pl.BlockSpec((D, B), lambda e, f: (0, 0)),
            pl.BlockSpec((1, C, D),
                         lambda e, f: (jnp.minimum(e, E - 1),
                                       jnp.where(e < E, f, NF - 1), 0)),
            pl.BlockSpec((1, C, D),
                         lambda e, f: (jnp.minimum(e, E - 1),
                                       jnp.where(e < E, f, NF - 1), 0)),
            pl.BlockSpec((1, DC, F),
                         lambda e, f: (jnp.maximum(e - 1, 0),
                                       jnp.where(e == 0, 0, f), 0)),
        ],
        out_specs=pl.BlockSpec((D, B), lambda e, f: (0, 0)),
        out_shape=jax.ShapeDtypeStruct((D, B), jnp.float32),
        scratch_shapes=[
            pltpu.VMEM((D, B), jnp.float32),
            pltpu.VMEM((E, B), jnp.float32),
            pltpu.VMEM((D, B), jnp.float32),
            pltpu.VMEM((F, B), jnp.float32),
            pltpu.VMEM((F, B), jnp.float32),
        ],
        compiler_params=pltpu.CompilerParams(
            dimension_semantics=("arbitrary", "arbitrary")),
    )(comb, curr.T, w1, w1_up, w2)
    return outt.T.reshape(x.shape[0], 1, D)


# speedup vs baseline: 1.0949x; 1.0949x over previous
import jax
import jax.numpy as jnp
from jax.experimental import pallas as pl
from jax.experimental.pallas import tpu as pltpu

E = 16
TOP_K = 8
D = 2048
F = 1408
B = 32
NF = 2
C = F // NF
DC = D // NF


def _ffn_kernel(rl_ref, xt_ref, w1_ref, w1u_ref, w2_ref, out_ref,
                xt_v, combt, acct, mixa, mixb):
    e = pl.program_id(0)
    f = pl.program_id(1)

    @pl.when((e == 0) & (f == 0))
    def _init():
        xt_v[:, :] = xt_ref[:, :]
        acct[:, :] = jnp.zeros((D, B), jnp.float32)
        logits = rl_ref[:, :]
        vals = logits
        sel = jnp.zeros((B, E), jnp.float32)
        idx = jax.lax.broadcasted_iota(jnp.int32, (B, E), 1)
        for _ in range(TOP_K):
            am = jnp.argmax(vals, axis=1)
            first = idx == am[:, None]
            sel = jnp.where(first, 1.0, sel)
            vals = jnp.where(first, -jnp.inf, vals)
        mx = jnp.max(logits, axis=1, keepdims=True)
        ew = jnp.exp(logits - mx) * sel
        w = ew / jnp.sum(ew, axis=1, keepdims=True)
        combt[:, :] = w.T

    @pl.when(e < E)
    def _gate_up():
        xtb = xt_v[:, :]
        gt = jax.lax.dot_general(w1_ref[0], xtb, (((1,), (0,)), ((), ())),
                                 preferred_element_type=jnp.float32)
        ut = jax.lax.dot_general(w1u_ref[0], xtb, (((1,), (0,)), ((), ())),
                                 preferred_element_type=jnp.float32)
        cw = combt[pl.ds(e, 1), :]
        mt = gt * jax.lax.logistic(gt) * ut * cw

        @pl.when(e % 2 == 0)
        def _():
            mixa[pl.ds(f * C, C), :] = mt

        @pl.when(e % 2 == 1)
        def _():
            mixb[pl.ds(f * C, C), :] = mt

    @pl.when(e > 0)
    def _down():
        @pl.when(e % 2 == 1)
        def _():
            acct[pl.ds(f * DC, DC), :] += jax.lax.dot_general(
                w2_ref[0], mixa[:, :], (((1,), (0,)), ((), ())),
                preferred_element_type=jnp.float32)

        @pl.when(e % 2 == 0)
        def _():
            acct[pl.ds(f * DC, DC), :] += jax.lax.dot_general(
                w2_ref[0], mixb[:, :], (((1,), (0,)), ((), ())),
                preferred_element_type=jnp.float32)

    @pl.when((e == E) & (f == NF - 1))
    def _fin():
        out_ref[:, :] = acct[:, :]


def kernel(x, router_logits, w1, w1_up, w2):
    if x.ndim == 2:
        x = x[:, None, :]
    curr = x[:, -1, :]
    outt = pl.pallas_call(
        _ffn_kernel,
        grid=(E + 1, NF),
        in_specs=[
            pl.BlockSpec((B, E), lambda e, f: (0, 0)),
            pl.BlockSpec((D, B), lambda e, f: (0, 0)),
            pl.BlockSpec((1, C, D),
                         lambda e, f: (jnp.minimum(e, E - 1),
                                       jnp.where(e < E, f, NF - 1), 0)),
            pl.BlockSpec((1, C, D),
                         lambda e, f: (jnp.minimum(e, E - 1),
                                       jnp.where(e < E, f, NF - 1), 0)),
            pl.BlockSpec((1, DC, F),
                         lambda e, f: (jnp.maximum(e - 1, 0),
                                       jnp.where(e == 0, 0, f), 0)),
        ],
        out_specs=pl.BlockSpec((D, B), lambda e, f: (0, 0)),
        out_shape=jax.ShapeDtypeStruct((D, B), jnp.float32),
        scratch_shapes=[
            pltpu.VMEM((D, B), jnp.float32),
            pltpu.VMEM((E, B), jnp.float32),
            pltpu.VMEM((D, B), jnp.float32),
            pltpu.VMEM((F, B), jnp.float32),
            pltpu.VMEM((F, B), jnp.float32),
        ],
        compiler_params=pltpu.CompilerParams(
            dimension_semantics=("arbitrary", "arbitrary")),
    )(router_logits, curr.T, w1, w1_up, w2)
    return outt.T.reshape(x.shape[0], 1, D)
